# baseline (speedup 1.0000x reference)
import functools

import jax
import jax.numpy as jnp
from jax import lax
from jax.experimental import pallas as pl
from jax.experimental.pallas import tpu as pltpu
from jax.experimental.pallas import tpu_sc as plsc

_L = 16
_P = 1024
_HALF = 512
_QC = _HALF // _L
_KC = _P // _L

_GATHER_DN = lax.GatherDimensionNumbers(
    offset_dims=(), collapsed_slice_dims=(0,), start_index_map=(0,)
)


def _take(vec, idx):
    return lax.gather(
        vec,
        idx[:, None],
        _GATHER_DN,
        slice_sizes=(1,),
        mode=lax.GatherScatterMode.PROMISE_IN_BOUNDS,
    )


def _sc_body(v_hbm, vp_hbm, m_hbm, pdw_hbm, out_hbm,
             q_raw, k_raw, m_raw, pdw_raw,
             qx, qy, qz, kx, ky, kz,
             kmin_ref, partner_ref, out_v, shared, *, inv_bp, inv_bpd):
    c = lax.axis_index("c")
    s = lax.axis_index("s")
    b = 8 * c + s // 2
    h = s % 2
    w = c * 16 + s
    lane = jnp.arange(_L, dtype=jnp.int32)

    pltpu.sync_copy(v_hbm.at[pl.ds(b * 3 * _P, 3 * _P)], k_raw)
    pltpu.sync_copy(
        vp_hbm.at[pl.ds(b * 3 * _P + h * 3 * _HALF, 3 * _HALF)], q_raw
    )
    pltpu.sync_copy(m_hbm.at[pl.ds(b * _P, _P)], m_raw)
    pltpu.sync_copy(
        pdw_hbm.at[pl.ds(b * 3 * _P + h * 3 * _HALF, 3 * _HALF)], pdw_raw
    )

    lane3 = lane * 3

    def plan_q(i, _):
        base = i * (_L * 3) + lane3
        qx[pl.ds(i * _L, _L)] = plsc.load_gather(q_raw, [base])
        qy[pl.ds(i * _L, _L)] = plsc.load_gather(q_raw, [base + 1])
        qz[pl.ds(i * _L, _L)] = plsc.load_gather(q_raw, [base + 2])
        return 0

    lax.fori_loop(0, _QC, plan_q, 0)

    def plan_k(i, _):
        base = i * (_L * 3) + lane3
        kx[pl.ds(i * _L, _L)] = plsc.load_gather(k_raw, [base])
        ky[pl.ds(i * _L, _L)] = plsc.load_gather(k_raw, [base + 1])
        kz[pl.ds(i * _L, _L)] = plsc.load_gather(k_raw, [base + 2])
        return 0

    lax.fori_loop(0, _KC, plan_k, 0)

    def pdw_loop(i, acc):
        x = pdw_raw[pl.ds(i * _L, _L)]
        return acc + x * x

    pdw_acc = lax.fori_loop(
        0, 3 * _HALF // _L, pdw_loop, jnp.zeros((_L,), jnp.float32)
    )

    big = jnp.full((_L,), 3.0e38, jnp.float32)

    def kc_loop(kc, qmins):
        kxc = kx[pl.ds(kc * _L, _L)]
        kyc = ky[pl.ds(kc * _L, _L)]
        kzc = kz[pl.ds(kc * _L, _L)]

        def r_loop(r, carry):
            kminv = carry[0]
            qm = list(carry[1:])
            idxf = (lane + r) & (_L - 1)
            idxb = (lane - r) & (_L - 1)
            kxr = _take(kxc, idxf)
            kyr = _take(kyc, idxf)
            kzr = _take(kzc, idxf)
            for qc in range(_QC):
                qxv = qx[pl.ds(qc * _L, _L)]
                qyv = qy[pl.ds(qc * _L, _L)]
                qzv = qz[pl.ds(qc * _L, _L)]
                dx = qxv - kxr
                dy = qyv - kyr
                dz = qzv - kzr
                d = dx * dx + dy * dy + dz * dz
                qm[qc] = jnp.minimum(qm[qc], d)
                dk = _take(d, idxb)
                kminv = jnp.minimum(kminv, dk)
            return (kminv, *qm)

        res = lax.fori_loop(0, _L, r_loop, (big, *qmins))
        kmin_ref[pl.ds(kc * _L, _L)] = res[0]
        return res[1:]

    qmins = lax.fori_loop(0, _KC, kc_loop, (big,) * _QC)

    qsum = jnp.zeros((_L,), jnp.float32)
    for qc in range(_QC):
        qsum = qsum + qmins[qc]

    pltpu.sync_copy(kmin_ref, shared.at[s])
    plsc.subcore_barrier()
    partial = qsum * inv_bp + pdw_acc * inv_bpd

    @pl.when(h == 0)
    def _():
        pltpu.sync_copy(shared.at[s + 1], partner_ref)

    def mask_loop(j, acc):
        mine = kmin_ref[pl.ds(j * _L, _L)]
        other = partner_ref[pl.ds(j * _L, _L)]
        merged = jnp.minimum(mine, other)
        mrow = m_raw[pl.ds(j * _L, _L)]
        return acc + merged * mrow

    masked = lax.fori_loop(0, _KC, mask_loop, jnp.zeros((_L,), jnp.float32))
    partial = jnp.where(h == 0, partial + masked * inv_bp, partial)

    out_v[...] = partial
    pltpu.sync_copy(out_v, out_hbm.at[pl.ds(w * _L, _L)])


def kernel(v, v_pred, mask, pred_dw):
    B, P, D = v.shape
    mesh = plsc.VectorSubcoreMesh(core_axis_name="c", subcore_axis_name="s")
    body = functools.partial(
        _sc_body, inv_bp=1.0 / (B * P), inv_bpd=1.0 / (B * P * D)
    )
    sc_fn = pl.kernel(
        body,
        mesh=mesh,
        compiler_params=pltpu.CompilerParams(needs_layout_passes=False),
        out_type=jax.ShapeDtypeStruct((32 * _L,), jnp.float32),
        scratch_types=[
            pltpu.VMEM((3 * _HALF,), jnp.float32),
            pltpu.VMEM((3 * _P,), jnp.float32),
            pltpu.VMEM((_P,), jnp.float32),
            pltpu.VMEM((3 * _HALF,), jnp.float32),
            pltpu.VMEM((_HALF,), jnp.float32),
            pltpu.VMEM((_HALF,), jnp.float32),
            pltpu.VMEM((_HALF,), jnp.float32),
            pltpu.VMEM((_P,), jnp.float32),
            pltpu.VMEM((_P,), jnp.float32),
            pltpu.VMEM((_P,), jnp.float32),
            pltpu.VMEM((_P,), jnp.float32),
            pltpu.VMEM((_P,), jnp.float32),
            pltpu.VMEM((_L,), jnp.float32),
            pltpu.VMEM_SHARED((16, _P), jnp.float32),
        ],
    )
    out = sc_fn(
        v.reshape(-1), v_pred.reshape(-1), mask.reshape(-1), pred_dw.reshape(-1)
    )
    return jnp.sum(out)


# speedup vs baseline: 3.2589x; 3.2589x over previous
import functools

import jax
import jax.numpy as jnp
from jax import lax
from jax.experimental import pallas as pl
from jax.experimental.pallas import tpu as pltpu
from jax.experimental.pallas import tpu_sc as plsc

_L = 16
_P = 1024
_NB = 2
_BPC = _NB // 2
_WPB = 16 // _BPC
_QW = _P // _WPB
_QCW = _QW // _L
_KC = _P // _L

_GATHER_DN = lax.GatherDimensionNumbers(
    offset_dims=(), collapsed_slice_dims=(0,), start_index_map=(0,)
)


def _take(vec, idx):
    return lax.gather(
        vec,
        idx[:, None],
        _GATHER_DN,
        slice_sizes=(1,),
        mode=lax.GatherScatterMode.PROMISE_IN_BOUNDS,
    )


def _sc_body(v_hbm, vp_hbm, m_hbm, pdw_hbm, out_hbm,
             q_raw, k_raw, m_raw, pdw_raw,
             qx, qy, qz, kx, ky, kz,
             kmin_ref, partner_ref, out_v, shared, *, inv_bp, inv_bpd):
    c = lax.axis_index("c")
    s = lax.axis_index("s")
    bl = s // _WPB
    b = _BPC * c + bl
    part = s % _WPB
    w = c * 16 + s
    lane = jnp.arange(_L, dtype=jnp.int32)

    q_off = b * 3 * _P + part * 3 * _QW
    pltpu.sync_copy(v_hbm.at[pl.ds(b * 3 * _P, 3 * _P)], k_raw)
    pltpu.sync_copy(vp_hbm.at[pl.ds(q_off, 3 * _QW)], q_raw)
    pltpu.sync_copy(m_hbm.at[pl.ds(b * _P, _P)], m_raw)
    pltpu.sync_copy(pdw_hbm.at[pl.ds(q_off, 3 * _QW)], pdw_raw)

    lane3 = lane * 3

    def plan_q(i, _):
        base = i * (_L * 3) + lane3
        qx[pl.ds(i * _L, _L)] = plsc.load_gather(q_raw, [base])
        qy[pl.ds(i * _L, _L)] = plsc.load_gather(q_raw, [base + 1])
        qz[pl.ds(i * _L, _L)] = plsc.load_gather(q_raw, [base + 2])
        return 0

    lax.fori_loop(0, _QCW, plan_q, 0)

    def plan_k(i, _):
        base = i * (_L * 3) + lane3
        kx[pl.ds(i * _L, _L)] = plsc.load_gather(k_raw, [base])
        ky[pl.ds(i * _L, _L)] = plsc.load_gather(k_raw, [base + 1])
        kz[pl.ds(i * _L, _L)] = plsc.load_gather(k_raw, [base + 2])
        return 0

    lax.fori_loop(0, _KC, plan_k, 0)

    def pdw_loop(i, acc):
        x = pdw_raw[pl.ds(i * _L, _L)]
        return acc + x * x

    pdw_acc = lax.fori_loop(
        0, 3 * _QW // _L, pdw_loop, jnp.zeros((_L,), jnp.float32)
    )

    big = jnp.full((_L,), 3.0e38, jnp.float32)
    idxf = [(lane + r) & (_L - 1) for r in range(_L)]
    idxb = [(lane - r) & (_L - 1) for r in range(_L)]

    def kc_loop(kc, qmins):
        kxc = kx[pl.ds(kc * _L, _L)]
        kyc = ky[pl.ds(kc * _L, _L)]
        kzc = kz[pl.ds(kc * _L, _L)]
        qm = list(qmins)
        kminv = big
        for r in range(_L):
            kxr = _take(kxc, idxf[r])
            kyr = _take(kyc, idxf[r])
            kzr = _take(kzc, idxf[r])
            for qc in range(_QCW):
                qxv = qx[pl.ds(qc * _L, _L)]
                qyv = qy[pl.ds(qc * _L, _L)]
                qzv = qz[pl.ds(qc * _L, _L)]
                dx = qxv - kxr
                dy = qyv - kyr
                dz = qzv - kzr
                d = dx * dx + dy * dy + dz * dz
                qm[qc] = jnp.minimum(qm[qc], d)
                kminv = jnp.minimum(kminv, _take(d, idxb[r]))
        kmin_ref[pl.ds(kc * _L, _L)] = kminv
        return tuple(qm)

    qmins = lax.fori_loop(0, _KC, kc_loop, (big,) * _QCW)

    qsum = jnp.zeros((_L,), jnp.float32)
    for qc in range(_QCW):
        qsum = qsum + qmins[qc]

    pltpu.sync_copy(kmin_ref, shared.at[s])
    plsc.subcore_barrier()
    partial = qsum * inv_bp + pdw_acc * inv_bpd

    for t in range(1, _WPB):
        pltpu.sync_copy(shared.at[bl * _WPB + t], partner_ref)

        def merge_loop(j, _):
            kmin_ref[pl.ds(j * _L, _L)] = jnp.minimum(
                kmin_ref[pl.ds(j * _L, _L)], partner_ref[pl.ds(j * _L, _L)]
            )
            return 0

        lax.fori_loop(0, _KC, merge_loop, 0)

    def mask_loop(j, acc):
        mrow = m_raw[pl.ds(j * _L, _L)]
        return acc + kmin_ref[pl.ds(j * _L, _L)] * mrow

    masked = lax.fori_loop(0, _KC, mask_loop, jnp.zeros((_L,), jnp.float32))
    partial = jnp.where(part == 0, partial + masked * inv_bp, partial)

    out_v[...] = partial
    pltpu.sync_copy(out_v, out_hbm.at[pl.ds(w * _L, _L)])


def _make_sc_fn(inv_bp, inv_bpd):
    mesh = plsc.VectorSubcoreMesh(core_axis_name="c", subcore_axis_name="s")
    body = functools.partial(_sc_body, inv_bp=inv_bp, inv_bpd=inv_bpd)
    return pl.kernel(
        body,
        mesh=mesh,
        compiler_params=pltpu.CompilerParams(needs_layout_passes=False),
        out_type=jax.ShapeDtypeStruct((32 * _L,), jnp.float32),
        scratch_types=[
            pltpu.VMEM((3 * _QW,), jnp.float32),
            pltpu.VMEM((3 * _P,), jnp.float32),
            pltpu.VMEM((_P,), jnp.float32),
            pltpu.VMEM((3 * _QW,), jnp.float32),
            pltpu.VMEM((_QW,), jnp.float32),
            pltpu.VMEM((_QW,), jnp.float32),
            pltpu.VMEM((_QW,), jnp.float32),
            pltpu.VMEM((_P,), jnp.float32),
            pltpu.VMEM((_P,), jnp.float32),
            pltpu.VMEM((_P,), jnp.float32),
            pltpu.VMEM((_P,), jnp.float32),
            pltpu.VMEM((_P,), jnp.float32),
            pltpu.VMEM((_L,), jnp.float32),
            pltpu.VMEM_SHARED((16, _P), jnp.float32),
        ],
    )


def _tc_kernel(q_ref, k_ref, m_ref, pdw_ref, out_ref, *, inv_bp, inv_bpd):
    b = pl.program_id(0)
    q = q_ref[0]
    k = k_ref[0]
    ones31 = jnp.ones((3, 1), jnp.float32)
    qq = jnp.dot(q * q, ones31, preferred_element_type=jnp.float32)
    k3 = k.T
    kk_row = jnp.sum(k3 * k3, axis=0, keepdims=True)
    a_aug = jnp.concatenate(
        [q * -2.0, qq, jnp.ones((1024, 1), jnp.float32)], axis=1
    )
    b_aug = jnp.concatenate(
        [k3, jnp.ones((1, 1024), jnp.float32), kk_row], axis=0
    )
    d = jnp.dot(a_aug, b_aug, preferred_element_type=jnp.float32)
    cham_x = jnp.min(d, axis=1)
    cham_y = jnp.min(d, axis=0)
    m = m_ref[0, 0, 0].reshape(1, 1024)
    pdw = pdw_ref[0]
    cham_y_masked = jnp.sum(cham_y.reshape(1, 1024) * m)
    part = (jnp.sum(cham_x) + cham_y_masked) * inv_bp
    part = part + jnp.sum(pdw * pdw) * inv_bpd

    @pl.when(b == 0)
    def _():
        out_ref[...] = jnp.zeros_like(out_ref)

    out_ref[...] += part[None, None]


def kernel(v, v_pred, mask, pred_dw):
    B, P, D = v.shape
    mb, mn, mc, mh, mw = mask.shape
    inv_bp = 1.0 / (B * P)
    inv_bpd = 1.0 / (B * P * D)

    sc_fn = _make_sc_fn(inv_bp, inv_bpd)
    sc_out = sc_fn(
        v.reshape(-1), v_pred.reshape(-1), mask.reshape(-1), pred_dw.reshape(-1)
    )

    kern = functools.partial(_tc_kernel, inv_bp=inv_bp, inv_bpd=inv_bpd)
    tc_out = pl.pallas_call(
        kern,
        grid=(B - _NB,),
        in_specs=[
            pl.BlockSpec((1, P, D), lambda b: (b + _NB, 0, 0)),
            pl.BlockSpec((1, P, D), lambda b: (b + _NB, 0, 0)),
            pl.BlockSpec(
                (1, 1, mc, mh, mw),
                lambda b: ((b + _NB) // mn, (b + _NB) % mn, 0, 0, 0),
            ),
            pl.BlockSpec((1, P, D), lambda b: (b + _NB, 0, 0)),
        ],
        out_specs=pl.BlockSpec((1, 1), lambda b: (0, 0)),
        out_shape=jax.ShapeDtypeStruct((1, 1), jnp.float32),
    )(v_pred, v, mask, pred_dw)

    return tc_out[0, 0] + jnp.sum(sc_out)


# speedup vs baseline: 8.4903x; 2.6053x over previous
import functools

import jax
import jax.numpy as jnp
from jax.experimental import pallas as pl


def _cch_kernel(q_ref, k_ref, m_ref, pdw_ref, out_ref, *, inv_bp, inv_bpd):
    b = pl.program_id(0)
    q = q_ref[0]
    k = k_ref[0]
    ones31 = jnp.ones((3, 1), jnp.float32)
    qq = jnp.dot(q * q, ones31, preferred_element_type=jnp.float32)
    k3 = k.T
    kk_row = jnp.sum(k3 * k3, axis=0, keepdims=True)
    a_aug = jnp.concatenate(
        [q * -2.0, qq, jnp.ones((1024, 1), jnp.float32)], axis=1
    )
    b_aug = jnp.concatenate(
        [k3, jnp.ones((1, 1024), jnp.float32), kk_row], axis=0
    )
    row_mins = []
    col_mins = []
    for j in range(4):
        d = jnp.dot(
            a_aug,
            b_aug[:, j * 256 : (j + 1) * 256],
            preferred_element_type=jnp.float32,
        )
        row_mins.append(jnp.min(d, axis=1, keepdims=True))
        col_mins.append(jnp.min(d, axis=0))
    cham_x = jnp.min(jnp.concatenate(row_mins, axis=1), axis=1)
    cham_y = jnp.concatenate(col_mins)
    m = m_ref[0, 0, 0].reshape(1, 1024)
    pdw = pdw_ref[0]
    cham_y_masked = jnp.sum(cham_y.reshape(1, 1024) * m)
    part = (jnp.sum(cham_x) + cham_y_masked) * inv_bp
    part = part + jnp.sum(pdw * pdw) * inv_bpd

    @pl.when(b == 0)
    def _():
        out_ref[...] = jnp.zeros_like(out_ref)

    out_ref[...] += part[None, None]


def kernel(v, v_pred, mask, pred_dw):
    B, P, D = v.shape
    mb, mn, mc, mh, mw = mask.shape
    kern = functools.partial(
        _cch_kernel, inv_bp=1.0 / (B * P), inv_bpd=1.0 / (B * P * D)
    )
    out = pl.pallas_call(
        kern,
        grid=(B,),
        in_specs=[
            pl.BlockSpec((1, P, D), lambda b: (b, 0, 0)),
            pl.BlockSpec((1, P, D), lambda b: (b, 0, 0)),
            pl.BlockSpec(
                (1, 1, mc, mh, mw), lambda b: (b // mn, b % mn, 0, 0, 0)
            ),
            pl.BlockSpec((1, P, D), lambda b: (b, 0, 0)),
        ],
        out_specs=pl.BlockSpec((1, 1), lambda b: (0, 0)),
        out_shape=jax.ShapeDtypeStruct((1, 1), jnp.float32),
    )(v_pred, v, mask, pred_dw)
    return out[0, 0]


# speedup vs baseline: 8.7770x; 1.0338x over previous
import functools

import jax
import jax.numpy as jnp
from jax.experimental import pallas as pl


def _cch_kernel(q_ref, k_ref, m_ref, pdw_ref, out_ref, *, inv_bp, inv_bpd):
    b = pl.program_id(0)
    q = q_ref[0]
    k = k_ref[0]
    ones31 = jnp.ones((3, 1), jnp.float32)
    qq = jnp.dot(q * q, ones31, preferred_element_type=jnp.float32)
    k3 = k.T
    kk_row = jnp.sum(k3 * k3, axis=0, keepdims=True)
    a_aug = jnp.concatenate(
        [q * -2.0, qq, jnp.ones((1024, 1), jnp.float32)], axis=1
    )
    b_aug = jnp.concatenate(
        [k3, jnp.ones((1, 1024), jnp.float32), kk_row], axis=0
    )
    d = jnp.dot(a_aug, b_aug, preferred_element_type=jnp.float32)
    cham_x = jnp.min(d, axis=1)
    cham_y = jnp.min(d, axis=0)
    m = m_ref[0, 0, 0].reshape(1, 1024)
    pdw = pdw_ref[0]
    cham_y_masked = jnp.sum(cham_y.reshape(1, 1024) * m)
    part = (jnp.sum(cham_x) + cham_y_masked) * inv_bp
    part = part + jnp.sum(pdw * pdw) * inv_bpd

    @pl.when(b == 0)
    def _():
        out_ref[...] = jnp.zeros_like(out_ref)

    out_ref[...] += part[None, None]


def kernel(v, v_pred, mask, pred_dw):
    B, P, D = v.shape
    mb, mn, mc, mh, mw = mask.shape
    kern = functools.partial(
        _cch_kernel, inv_bp=1.0 / (B * P), inv_bpd=1.0 / (B * P * D)
    )
    out = pl.pallas_call(
        kern,
        grid=(B,),
        in_specs=[
            pl.BlockSpec((1, P, D), lambda b: (b, 0, 0)),
            pl.BlockSpec((1, P, D), lambda b: (b, 0, 0)),
            pl.BlockSpec(
                (1, 1, mc, mh, mw), lambda b: (b // mn, b % mn, 0, 0, 0)
            ),
            pl.BlockSpec((1, P, D), lambda b: (b, 0, 0)),
        ],
        out_specs=pl.BlockSpec((1, 1), lambda b: (0, 0)),
        out_shape=jax.ShapeDtypeStruct((1, 1), jnp.float32),
    )(v_pred, v, mask, pred_dw)
    return out[0, 0]
